# baseline (speedup 1.0000x reference)
import jax
import jax.numpy as jnp
from jax.experimental import pallas as pl
from jax.experimental.pallas import tpu as pltpu

_H0, _W0 = 56, 120
_N_CAM, _M_HEADS, _HG, _WG = 6, 4, 25, 25
_P = _HG * _WG
_K = _N_CAM * _H0 * _W0
_PB = 5
_NRB = _P // _PB
_STEPS = (_NRB + 1) // 2


def _argmax_body(att_ref, ood_ref, val_ref):
    x = att_ref[...].reshape(_PB, 8, _K)
    ki = jax.lax.broadcasted_iota(jnp.int32, (_PB, _K), 1)

    for b in range(2):
        s = (x[:, 4 * b] + x[:, 4 * b + 1]
             + x[:, 4 * b + 2] + x[:, 4 * b + 3])
        bmax = jnp.max(s, axis=-1, keepdims=True)
        cand = jnp.where(s == bmax, ki, _K)
        bidx = jnp.min(cand, axis=-1, keepdims=True)
        ood_b = ood_ref[b]
        bval = jnp.sum(jnp.where(ki == bidx, ood_b, 0.0),
                       axis=-1)
        val_ref[0, b] = bval


def _bce_body(mask_ref, y_ref, out_ref):
    m = mask_ref[0]
    t = y_ref[0, 0]
    logp = jnp.maximum(jnp.log(m), -100.0)
    log1mp = jnp.maximum(jnp.log1p(-m), -100.0)

    cell = jax.lax.broadcasted_iota(jnp.int32, (_WG, 8 * _WG), 1) // 8
    row = jax.lax.broadcasted_iota(jnp.int32, (_WG, 8 * _WG), 0)
    g = (cell == row).astype(jnp.float32)
    logp_w = jnp.dot(logp, g, preferred_element_type=jnp.float32)
    log1mp_w = jnp.dot(log1mp, g, preferred_element_type=jnp.float32)
    tc = jnp.dot(g, t, preferred_element_type=jnp.float32)
    out_ref[...] = -(jnp.sum(tc * logp_w)
                     + jnp.sum((8.0 - tc) * log1mp_w))[None, None, None]


def kernel(alpha, y, ood, ood_cam, att0, att1):
    B = y.shape[0]
    att2d = att0.transpose(1, 0, 2).reshape(_P * 2 * _M_HEADS, _K)
    ood_flat = ood_cam.reshape(B, 1, _K)

    val = pl.pallas_call(
        _argmax_body,
        grid=(2, _STEPS),
        in_specs=[
            pl.BlockSpec((_PB * 8, _K),
                         lambda c, k: (jnp.minimum(c * _STEPS + k,
                                                   _NRB - 1), 0)),
            pl.BlockSpec((B, 1, _K), lambda c, k: (0, 0, 0)),
        ],
        out_specs=pl.BlockSpec((1, B, _PB),
                               lambda c, k: (jnp.minimum(c * _STEPS + k,
                                                         _NRB - 1), 0, 0)),
        out_shape=jax.ShapeDtypeStruct((_NRB, B, _PB), jnp.float32),
        compiler_params=pltpu.CompilerParams(
            dimension_semantics=("parallel", "arbitrary"),
            vmem_limit_bytes=56 * 1024 * 1024,
        ),
    )(att2d, ood_flat)

    mask = val.transpose(1, 0, 2).reshape(B, _HG, _WG)

    out = pl.pallas_call(
        _bce_body,
        grid=(B,),
        in_specs=[
            pl.BlockSpec((1, _HG, _WG), lambda b: (b, 0, 0)),
            pl.BlockSpec((1, 1, 8 * _HG, 8 * _WG), lambda b: (b, 0, 0, 0)),
        ],
        out_specs=pl.BlockSpec((1, 1, 1), lambda b: (b, 0, 0)),
        out_shape=jax.ShapeDtypeStruct((B, 1, 1), jnp.float32),
        compiler_params=pltpu.CompilerParams(
            dimension_semantics=("parallel",),
        ),
    )(mask, y)

    return out.sum() / (B * 8 * _HG * 8 * _WG)


# speedup vs baseline: 3.4187x; 3.4187x over previous
import jax
import jax.numpy as jnp
from jax.experimental import pallas as pl
from jax.experimental.pallas import tpu as pltpu

_H0, _W0 = 56, 120
_N_CAM, _M_HEADS, _HG, _WG = 6, 4, 25, 25
_P = _HG * _WG
_K = _N_CAM * _H0 * _W0
_PB = 5
_NRB = _P // _PB
_STEPS = (_NRB + 1) // 2


def _argmax_body(att_ref, ood_ref, val_ref):
    x = att_ref[...]
    rr = jax.lax.broadcasted_iota(jnp.int32, (16, _PB * 8), 0)
    cc = jax.lax.broadcasted_iota(jnp.int32, (16, _PB * 8), 1)
    sel = ((cc // 8 == rr % 8) & ((cc % 8) // 4 == rr // 8)
           & (rr % 8 < _PB)).astype(jnp.float32)
    s16 = jnp.dot(sel, x, preferred_element_type=jnp.float32)

    ki = jax.lax.broadcasted_iota(jnp.int32, (8, _K), 1)
    for b in range(2):
        s = s16[8 * b:8 * b + 8]
        bmax = jnp.max(s, axis=-1, keepdims=True)
        cand = jnp.where(s == bmax, ki, _K)
        bidx = jnp.min(cand, axis=-1, keepdims=True)
        ood_b = ood_ref[b]
        bval = jnp.sum(jnp.where(ki == bidx, ood_b, 0.0),
                       axis=-1)
        val_ref[0, b] = bval[0:_PB]


def _bce_body(mask_ref, y_ref, out_ref):
    m = mask_ref[0]
    t = y_ref[0, 0]
    logp = jnp.maximum(jnp.log(m), -100.0)
    log1mp = jnp.maximum(jnp.log1p(-m), -100.0)

    cell = jax.lax.broadcasted_iota(jnp.int32, (_WG, 8 * _WG), 1) // 8
    row = jax.lax.broadcasted_iota(jnp.int32, (_WG, 8 * _WG), 0)
    g = (cell == row).astype(jnp.float32)
    logp_w = jnp.dot(logp, g, preferred_element_type=jnp.float32)
    log1mp_w = jnp.dot(log1mp, g, preferred_element_type=jnp.float32)
    tc = jnp.dot(g, t, preferred_element_type=jnp.float32)
    out_ref[...] = -(jnp.sum(tc * logp_w)
                     + jnp.sum((8.0 - tc) * log1mp_w))[None, None, None]


def kernel(alpha, y, ood, ood_cam, att0, att1):
    B = y.shape[0]
    att2d = att0.transpose(1, 0, 2).reshape(_P * 2 * _M_HEADS, _K)
    ood_flat = ood_cam.reshape(B, 1, _K)

    val = pl.pallas_call(
        _argmax_body,
        grid=(2, _STEPS),
        in_specs=[
            pl.BlockSpec((_PB * 8, _K),
                         lambda c, k: (jnp.minimum(c * _STEPS + k,
                                                   _NRB - 1), 0)),
            pl.BlockSpec((B, 1, _K), lambda c, k: (0, 0, 0)),
        ],
        out_specs=pl.BlockSpec((1, B, _PB),
                               lambda c, k: (jnp.minimum(c * _STEPS + k,
                                                         _NRB - 1), 0, 0)),
        out_shape=jax.ShapeDtypeStruct((_NRB, B, _PB), jnp.float32),
        compiler_params=pltpu.CompilerParams(
            dimension_semantics=("parallel", "arbitrary"),
            vmem_limit_bytes=56 * 1024 * 1024,
        ),
    )(att2d, ood_flat)

    mask = val.transpose(1, 0, 2).reshape(B, _HG, _WG)

    out = pl.pallas_call(
        _bce_body,
        grid=(B,),
        in_specs=[
            pl.BlockSpec((1, _HG, _WG), lambda b: (b, 0, 0)),
            pl.BlockSpec((1, 1, 8 * _HG, 8 * _WG), lambda b: (b, 0, 0, 0)),
        ],
        out_specs=pl.BlockSpec((1, 1, 1), lambda b: (b, 0, 0)),
        out_shape=jax.ShapeDtypeStruct((B, 1, 1), jnp.float32),
        compiler_params=pltpu.CompilerParams(
            dimension_semantics=("parallel",),
        ),
    )(mask, y)

    return out.sum() / (B * 8 * _HG * 8 * _WG)


# speedup vs baseline: 4.1079x; 1.2016x over previous
import jax
import jax.numpy as jnp
from jax.experimental import pallas as pl
from jax.experimental.pallas import tpu as pltpu

_H0, _W0 = 56, 120
_N_CAM, _M_HEADS, _HG, _WG = 6, 4, 25, 25
_P = _HG * _WG
_K = _N_CAM * _H0 * _W0
_PB = 5
_NRB = _P // _PB
_NT = (_NRB + 2) // 3
_TSTEPS = _NT // 2


def _argmax_body(a0_ref, a1_ref, a2_ref, ood_ref, val_ref):
    rr = jax.lax.broadcasted_iota(jnp.int32, (16, _PB * 8), 0)
    cc = jax.lax.broadcasted_iota(jnp.int32, (16, _PB * 8), 1)
    sel = ((cc // 8 == rr % 8) & ((cc % 8) // 4 == rr // 8)
           & (rr % 8 < _PB)).astype(jnp.float32)
    ki = jax.lax.broadcasted_iota(jnp.int32, (8, _K), 1)

    for j, att_ref in enumerate((a0_ref, a1_ref, a2_ref)):
        x = att_ref[...]
        s16 = jnp.dot(sel, x, preferred_element_type=jnp.float32)
        for b in range(2):
            s = s16[8 * b:8 * b + 8]
            bmax = jnp.max(s, axis=-1, keepdims=True)
            cand = jnp.where(s == bmax, ki, _K)
            bidx = jnp.min(cand, axis=-1, keepdims=True)
            ood_b = ood_ref[b]
            bval = jnp.sum(jnp.where(ki == bidx, ood_b, 0.0),
                           axis=-1)
            val_ref[j, b] = bval[0:_PB]


def _bce_body(mask_ref, y_ref, out_ref):
    m = mask_ref[0]
    t = y_ref[0, 0]
    logp = jnp.maximum(jnp.log(m), -100.0)
    log1mp = jnp.maximum(jnp.log1p(-m), -100.0)

    cell = jax.lax.broadcasted_iota(jnp.int32, (_WG, 8 * _WG), 1) // 8
    row = jax.lax.broadcasted_iota(jnp.int32, (_WG, 8 * _WG), 0)
    g = (cell == row).astype(jnp.float32)
    logp_w = jnp.dot(logp, g, preferred_element_type=jnp.float32)
    log1mp_w = jnp.dot(log1mp, g, preferred_element_type=jnp.float32)
    tc = jnp.dot(g, t, preferred_element_type=jnp.float32)
    out_ref[...] = -(jnp.sum(tc * logp_w)
                     + jnp.sum((8.0 - tc) * log1mp_w))[None, None, None]


def kernel(alpha, y, ood, ood_cam, att0, att1):
    B = y.shape[0]
    att2d = att0.transpose(1, 0, 2).reshape(_P * 2 * _M_HEADS, _K)
    ood_flat = ood_cam.reshape(B, 1, _K)

    val = pl.pallas_call(
        _argmax_body,
        grid=(2, _TSTEPS),
        in_specs=[
            pl.BlockSpec((_PB * 8, _K),
                         (lambda c, k, j=j: (
                             jnp.minimum(3 * jnp.minimum(c * _TSTEPS + k,
                                                         _NT - 1) + j,
                                         _NRB - 1), 0)))
            for j in range(3)
        ] + [
            pl.BlockSpec((B, 1, _K), lambda c, k: (0, 0, 0)),
        ],
        out_specs=pl.BlockSpec((3, B, _PB),
                               lambda c, k: (jnp.minimum(c * _TSTEPS + k,
                                                         _NT - 1), 0, 0)),
        out_shape=jax.ShapeDtypeStruct((3 * _NT, B, _PB), jnp.float32),
        compiler_params=pltpu.CompilerParams(
            dimension_semantics=("parallel", "arbitrary"),
            vmem_limit_bytes=56 * 1024 * 1024,
        ),
    )(att2d, att2d, att2d, ood_flat)

    mask = val[:_NRB].transpose(1, 0, 2).reshape(B, _HG, _WG)

    out = pl.pallas_call(
        _bce_body,
        grid=(B,),
        in_specs=[
            pl.BlockSpec((1, _HG, _WG), lambda b: (b, 0, 0)),
            pl.BlockSpec((1, 1, 8 * _HG, 8 * _WG), lambda b: (b, 0, 0, 0)),
        ],
        out_specs=pl.BlockSpec((1, 1, 1), lambda b: (b, 0, 0)),
        out_shape=jax.ShapeDtypeStruct((B, 1, 1), jnp.float32),
        compiler_params=pltpu.CompilerParams(
            dimension_semantics=("parallel",),
        ),
    )(mask, y)

    return out.sum() / (B * 8 * _HG * 8 * _WG)
